# baseline (speedup 1.0000x reference)
import functools

import numpy as np

import jax
import jax.numpy as jnp
from jax import lax
from jax.experimental import pallas as pl
from jax.experimental.pallas import tpu as pltpu
from jax._src.random.threefry2x32 import threefry2x32_p as _threefry2x32_p

_D = 64
_S = 10
_TILE = 64

_KS0 = np.uint32(0)
_KS1 = np.uint32(42)
_KS2 = np.uint32(_KS0 ^ _KS1 ^ np.uint32(0x1BD11BDA))
_ROT_A = (13, 15, 26, 6)
_ROT_B = (17, 29, 16, 24)
_INJ = (
    (_KS1, np.uint32(_KS2 + np.uint32(1))),
    (_KS2, np.uint32(_KS0 + np.uint32(2))),
    (_KS0, np.uint32(_KS1 + np.uint32(3))),
    (_KS1, np.uint32(_KS2 + np.uint32(4))),
    (_KS2, np.uint32(_KS0 + np.uint32(5))),
)
_TINY = np.float32(np.finfo(np.float32).tiny)


def _rotl(v, r):
    return lax.shift_left(v, np.uint32(r)) | lax.shift_right_logical(
        v, np.uint32(32 - r))


def _threefry_bits(cnt):
    k1 = jnp.full((1, 1), _KS0, jnp.uint32)
    k2 = jnp.full((1, 1), _KS1, jnp.uint32)
    o1, o2 = _threefry2x32_p.bind(k1, k2, jnp.zeros_like(cnt), cnt)
    return o1 ^ o2


def _vq_body(n, K, x_ref, e_ref, q_ref, s_ref):
    i = pl.program_id(0)
    x = x_ref[...]
    e = e_ref[...]
    prod = lax.dot_general(x, e, (((1,), (1,)), ((), ())),
                           preferred_element_type=jnp.float32)
    xn = jnp.sum(x * x, axis=1, keepdims=True)
    ones = jnp.ones((1, _D), jnp.float32)
    en = lax.dot_general(ones, e * e, (((1,), (1,)), ((), ())),
                         preferred_element_type=jnp.float32)
    logits = -((xn + en - (prod + prod)) * np.float32(1.0 / K))

    col = lax.broadcasted_iota(jnp.int32, (_TILE, K), 1)
    rowu = lax.broadcasted_iota(jnp.uint32, (_TILE, K), 0)
    iu = lax.convert_element_type(i, jnp.uint32)
    base = (iu * np.uint32(_TILE * K)
            + rowu * np.uint32(K)
            + col.astype(jnp.uint32))

    idxs = []
    counts = jnp.zeros((_TILE, K), jnp.float32)
    for s in range(_S):
        cnt = base + np.uint32(s * n * K)
        bits = _threefry_bits(cnt)
        fb = lax.shift_right_logical(bits, np.uint32(9)) | np.uint32(0x3F800000)
        f = lax.bitcast_convert_type(fb, jnp.float32) - np.float32(1.0)
        u = jnp.maximum(f + _TINY, _TINY)
        v = -jnp.log(-jnp.log(u)) + logits
        mx = jnp.max(v, axis=1, keepdims=True)
        mask = v == mx
        idx = jnp.min(jnp.where(mask, col, np.int32(K)), axis=1)
        idxs.append(idx[:, None])
        counts = counts + mask.astype(jnp.float32)
    samples = jnp.concatenate(idxs, axis=1)
    s_ref[...] = samples
    q = lax.dot_general(counts, e, (((1,), (0,)), ((), ())),
                        preferred_element_type=jnp.float32) / np.float32(_S)
    q_ref[...] = x + (q - x)


def kernel(inputs, embeds):
    size = inputs.shape
    x = inputs.reshape(-1, _D)
    n = x.shape[0]
    K = embeds.shape[0]
    grid = (n // _TILE,)
    body = functools.partial(_vq_body, n, K)
    q, s = pl.pallas_call(
        body,
        grid=grid,
        in_specs=[
            pl.BlockSpec((_TILE, _D), lambda i: (i, 0)),
            pl.BlockSpec((K, _D), lambda i: (0, 0)),
        ],
        out_specs=[
            pl.BlockSpec((_TILE, _D), lambda i: (i, 0)),
            pl.BlockSpec((_TILE, _S), lambda i: (i, 0)),
        ],
        out_shape=[
            jax.ShapeDtypeStruct((n, _D), jnp.float32),
            jax.ShapeDtypeStruct((n, _S), jnp.int32),
        ],
        compiler_params=pltpu.CompilerParams(
            dimension_semantics=("parallel",)),
    )(x, embeds)
    return (q.reshape(size), s.reshape(size[:-1] + (_S,)))


# speedup vs baseline: 1.0303x; 1.0303x over previous
import functools

import numpy as np

import jax
import jax.numpy as jnp
from jax import lax
from jax.experimental import pallas as pl
from jax.experimental.pallas import tpu as pltpu
from jax.experimental.pallas import tpu_sc as plsc

_D = 64
_S = 10
_TILE = 64

_KS0 = np.uint32(0)
_KS1 = np.uint32(42)
_KS2 = np.uint32(_KS0 ^ _KS1 ^ np.uint32(0x1BD11BDA))
_ROT_A = (13, 15, 26, 6)
_ROT_B = (17, 29, 16, 24)
_INJ = (
    (_KS1, np.uint32(_KS2 + np.uint32(1))),
    (_KS2, np.uint32(_KS0 + np.uint32(2))),
    (_KS0, np.uint32(_KS1 + np.uint32(3))),
    (_KS1, np.uint32(_KS2 + np.uint32(4))),
    (_KS2, np.uint32(_KS0 + np.uint32(5))),
)
_TINY = np.float32(np.finfo(np.float32).tiny)

_NC = 2
_NS = 16
_NW = _NC * _NS
_L = 16
_CHUNK = 120


def _rotl(v, r):
    return lax.shift_left(v, np.uint32(r)) | lax.shift_right_logical(
        v, np.uint32(32 - r))


def _threefry_bits(cnt):
    x0 = jnp.full_like(cnt, _KS0)
    x1 = cnt + _KS1
    for g in range(5):
        rots = _ROT_A if g % 2 == 0 else _ROT_B
        for r in rots:
            x0 = x0 + x1
            x1 = _rotl(x1, r)
            x1 = x1 ^ x0
        a, b = _INJ[g]
        x0 = x0 + a
        x1 = x1 + b
    return x0 ^ x1


def _sample_body(n, K, x_ref, e_ref, s_ref):
    i = pl.program_id(0)
    x = x_ref[...]
    e = e_ref[...]
    prod = lax.dot_general(x, e, (((1,), (1,)), ((), ())),
                           preferred_element_type=jnp.float32)
    xn = jnp.sum(x * x, axis=1, keepdims=True)
    ones = jnp.ones((1, _D), jnp.float32)
    en = lax.dot_general(ones, e * e, (((1,), (1,)), ((), ())),
                         preferred_element_type=jnp.float32)
    logits = -((xn + en - (prod + prod)) * np.float32(1.0 / K))

    col = lax.broadcasted_iota(jnp.int32, (_TILE, K), 1)
    rowu = lax.broadcasted_iota(jnp.uint32, (_TILE, K), 0)
    iu = lax.convert_element_type(i, jnp.uint32)
    base = (iu * np.uint32(_TILE * K)
            + rowu * np.uint32(K)
            + col.astype(jnp.uint32))

    idxs = []
    for s in range(_S):
        cnt = base + np.uint32(s * n * K)
        bits = _threefry_bits(cnt)
        fb = lax.shift_right_logical(bits, np.uint32(9)) | np.uint32(0x3F800000)
        f = lax.bitcast_convert_type(fb, jnp.float32) - np.float32(1.0)
        u = jnp.maximum(f + _TINY, _TINY)
        v = -jnp.log(-jnp.log(u)) + logits
        mx = jnp.max(v, axis=1, keepdims=True)
        mask = v == mx
        idx = jnp.min(jnp.where(mask, col, np.int32(K)), axis=1)
        idxs.append(idx[:, None])
    s_ref[...] = jnp.concatenate(idxs, axis=1)


def _tc_sample(x, embeds):
    n = x.shape[0]
    K = embeds.shape[0]
    body = functools.partial(_sample_body, n, K)
    return pl.pallas_call(
        body,
        grid=(n // _TILE,),
        in_specs=[
            pl.BlockSpec((_TILE, _D), lambda i: (i, 0)),
            pl.BlockSpec((K, _D), lambda i: (0, 0)),
        ],
        out_specs=pl.BlockSpec((_TILE, _S), lambda i: (i, 0)),
        out_shape=jax.ShapeDtypeStruct((n, _S), jnp.int32),
        compiler_params=pltpu.CompilerParams(
            dimension_semantics=("parallel",)),
    )(x, embeds)


def _sc_gather_mean(x, table128, samples_idx):
    n = x.shape[0]
    tok_per_w = n // _NW
    b_per_w = tok_per_w * _S
    n_chunks = b_per_w // _CHUNK
    blk_chunks = 4
    blk_samp = blk_chunks * _CHUNK
    blk_tok = blk_samp // _S
    n_blocks = n_chunks // blk_chunks
    mesh = plsc.VectorSubcoreMesh(core_axis_name="c", subcore_axis_name="s")

    @functools.partial(
        pl.kernel, mesh=mesh,
        out_type=jax.ShapeDtypeStruct((n, _D), jnp.float32),
        scratch_types=[
            pltpu.VMEM((n_chunks, _CHUNK), jnp.int32),
            pltpu.VMEM((blk_samp, 2 * _D), jnp.float32),
            pltpu.VMEM((tok_per_w, _D), jnp.float32),
            pltpu.SemaphoreType.DMA,
        ],
    )
    def k(x_hbm, table_hbm, idx_hbm, out_hbm, idx_v, rows_v, xq_v, sem):
        wid = lax.axis_index("s") * _NC + lax.axis_index("c")
        tbase = wid * tok_per_w
        pltpu.sync_copy(idx_hbm.at[wid], idx_v)
        pltpu.sync_copy(x_hbm.at[pl.ds(tbase, tok_per_w)], xq_v)
        for b in range(n_blocks):
            for j in range(blk_chunks):
                pltpu.async_copy(
                    table_hbm.at[idx_v.at[b * blk_chunks + j]],
                    rows_v.at[pl.ds(j * _CHUNK, _CHUNK)], sem)
            for j in range(blk_chunks):
                pltpu.make_async_copy(
                    table_hbm.at[idx_v.at[b * blk_chunks + j]],
                    rows_v.at[pl.ds(j * _CHUNK, _CHUNK)], sem).wait()

            def body(t, carry):
                for vv in range(_D // _L):
                    sl = pl.ds(vv * _L, _L)
                    acc = rows_v[t * _S + 0, sl]
                    for s in range(1, _S):
                        acc = acc + rows_v[t * _S + s, sl]
                    q = acc / np.float32(_S)
                    tt = b * blk_tok + t
                    xv = xq_v[tt, sl]
                    xq_v[tt, sl] = xv + (q - xv)
                return carry

            lax.fori_loop(0, blk_tok, body, 0)
        pltpu.sync_copy(xq_v, out_hbm.at[pl.ds(tbase, tok_per_w)])

    return k(x, table128, samples_idx)


def kernel(inputs, embeds):
    size = inputs.shape
    x = inputs.reshape(-1, _D)
    samples = _tc_sample(x, embeds)
    table128 = jnp.pad(embeds, ((0, 0), (0, _D)))
    q = _sc_gather_mean(x, table128,
                        samples.reshape(_NW, -1, _CHUNK))
    return (q.reshape(size), samples.reshape(size[:-1] + (_S,)))


# speedup vs baseline: 1.0703x; 1.0388x over previous
import functools

import numpy as np

import jax
import jax.numpy as jnp
from jax import lax
from jax.experimental import pallas as pl
from jax.experimental.pallas import tpu as pltpu
from jax.experimental.pallas import tpu_sc as plsc

_D = 64
_S = 10
_TILE = 128

_KS0 = np.uint32(0)
_KS1 = np.uint32(42)
_KS2 = np.uint32(_KS0 ^ _KS1 ^ np.uint32(0x1BD11BDA))
_ROT_A = (13, 15, 26, 6)
_ROT_B = (17, 29, 16, 24)
_INJ = (
    (_KS1, np.uint32(_KS2 + np.uint32(1))),
    (_KS2, np.uint32(_KS0 + np.uint32(2))),
    (_KS0, np.uint32(_KS1 + np.uint32(3))),
    (_KS1, np.uint32(_KS2 + np.uint32(4))),
    (_KS2, np.uint32(_KS0 + np.uint32(5))),
)
_TINY = np.float32(np.finfo(np.float32).tiny)

_NC = 2
_NS = 16
_NW = _NC * _NS
_L = 16
_CHUNK = 120


def _rotl_i32(v, r):
    return lax.shift_left(v, np.int32(r)) | lax.shift_right_logical(
        v, np.int32(32 - r))


def _threefry_bits(cnt):
    x1 = cnt + np.int32(_KS1)
    x0 = x1
    first = True
    for g in range(5):
        rots = _ROT_A if g % 2 == 0 else _ROT_B
        for r in rots:
            if first:
                first = False
            else:
                x0 = x0 + x1
            x1 = _rotl_i32(x1, r)
            x1 = x1 ^ x0
        a, b = _INJ[g]
        x0 = x0 + np.int32(a)
        x1 = x1 + np.int32(b)
    return (x0 ^ x1).astype(jnp.uint32)


def _sample_body(n, K, x_ref, e_ref, s_ref):
    i = pl.program_id(0)
    x = x_ref[...]
    e = e_ref[...]
    prod = lax.dot_general(x, e, (((1,), (1,)), ((), ())),
                           preferred_element_type=jnp.float32)
    xn = jnp.sum(x * x, axis=1, keepdims=True)
    ones = jnp.ones((1, _D), jnp.float32)
    en = lax.dot_general(ones, e * e, (((1,), (1,)), ((), ())),
                         preferred_element_type=jnp.float32)
    logits = -((xn + en - (prod + prod)) * np.float32(1.0 / K))

    col = lax.broadcasted_iota(jnp.int32, (_TILE, K), 1)
    row = lax.broadcasted_iota(jnp.int32, (_TILE, K), 0)
    base = i * np.int32(_TILE * K) + row * np.int32(K) + col

    idxs = []
    for s in range(_S):
        cnt = base + np.int32(s * n * K)
        bits = _threefry_bits(cnt)
        fb = lax.shift_right_logical(bits, np.uint32(9)) | np.uint32(0x3F800000)
        f = lax.bitcast_convert_type(fb, jnp.float32) - np.float32(1.0)
        u = jnp.maximum(f + _TINY, _TINY)
        v = -jnp.log(-jnp.log(u)) + logits
        mx = jnp.max(v, axis=1, keepdims=True)
        mask = v == mx
        idx = jnp.min(jnp.where(mask, col, np.int32(K)), axis=1)
        idxs.append(idx[:, None])
    s_ref[...] = jnp.concatenate(idxs, axis=1)


def _tc_sample(x, embeds):
    n = x.shape[0]
    K = embeds.shape[0]
    body = functools.partial(_sample_body, n, K)
    return pl.pallas_call(
        body,
        grid=(n // _TILE,),
        in_specs=[
            pl.BlockSpec((_TILE, _D), lambda i: (i, 0)),
            pl.BlockSpec((K, _D), lambda i: (0, 0)),
        ],
        out_specs=pl.BlockSpec((_TILE, _S), lambda i: (i, 0)),
        out_shape=jax.ShapeDtypeStruct((n, _S), jnp.int32),
        compiler_params=pltpu.CompilerParams(
            dimension_semantics=("parallel",)),
    )(x, embeds)


def _sc_gather_mean(x, table128, samples_idx):
    n = x.shape[0]
    tok_per_w = n // _NW
    b_per_w = tok_per_w * _S
    n_chunks = b_per_w // _CHUNK
    blk_chunks = 4
    blk_samp = blk_chunks * _CHUNK
    blk_tok = blk_samp // _S
    n_blocks = n_chunks // blk_chunks
    mesh = plsc.VectorSubcoreMesh(core_axis_name="c", subcore_axis_name="s")

    @functools.partial(
        pl.kernel, mesh=mesh,
        out_type=jax.ShapeDtypeStruct((n, _D), jnp.float32),
        scratch_types=[
            pltpu.VMEM((n_chunks, _CHUNK), jnp.int32),
            pltpu.VMEM((blk_samp, 2 * _D), jnp.float32),
            pltpu.VMEM((tok_per_w, _D), jnp.float32),
            pltpu.SemaphoreType.DMA,
        ],
    )
    def k(x_hbm, table_hbm, idx_hbm, out_hbm, idx_v, rows_v, xq_v, sem):
        wid = lax.axis_index("s") * _NC + lax.axis_index("c")
        tbase = wid * tok_per_w
        pltpu.sync_copy(idx_hbm.at[wid], idx_v)
        pltpu.sync_copy(x_hbm.at[pl.ds(tbase, tok_per_w)], xq_v)
        for b in range(n_blocks):
            for j in range(blk_chunks):
                pltpu.async_copy(
                    table_hbm.at[idx_v.at[b * blk_chunks + j]],
                    rows_v.at[pl.ds(j * _CHUNK, _CHUNK)], sem)
            for j in range(blk_chunks):
                pltpu.make_async_copy(
                    table_hbm.at[idx_v.at[b * blk_chunks + j]],
                    rows_v.at[pl.ds(j * _CHUNK, _CHUNK)], sem).wait()

            def body(t, carry):
                for vv in range(_D // _L):
                    sl = pl.ds(vv * _L, _L)
                    acc = rows_v[t * _S + 0, sl]
                    for s in range(1, _S):
                        acc = acc + rows_v[t * _S + s, sl]
                    q = acc / np.float32(_S)
                    tt = b * blk_tok + t
                    xv = xq_v[tt, sl]
                    xq_v[tt, sl] = xv + (q - xv)
                return carry

            lax.fori_loop(0, blk_tok, body, 0)
        pltpu.sync_copy(xq_v, out_hbm.at[pl.ds(tbase, tok_per_w)])

    return k(x, table128, samples_idx)


def kernel(inputs, embeds):
    size = inputs.shape
    x = inputs.reshape(-1, _D)
    samples = _tc_sample(x, embeds)
    table128 = jnp.pad(embeds, ((0, 0), (0, _D)))
    q = _sc_gather_mean(x, table128,
                        samples.reshape(_NW, -1, _CHUNK))
    return (q.reshape(size), samples.reshape(size[:-1] + (_S,)))


# speedup vs baseline: 1.0887x; 1.0172x over previous
import functools

import numpy as np

import jax
import jax.numpy as jnp
from jax import lax
from jax.experimental import pallas as pl
from jax.experimental.pallas import tpu as pltpu
from jax.experimental.pallas import tpu_sc as plsc

_D = 64
_S = 10
_TILE = 128

_KS0 = np.uint32(0)
_KS1 = np.uint32(42)
_KS2 = np.uint32(_KS0 ^ _KS1 ^ np.uint32(0x1BD11BDA))
_ROT_A = (13, 15, 26, 6)
_ROT_B = (17, 29, 16, 24)
_INJ = (
    (_KS1, np.uint32(_KS2 + np.uint32(1))),
    (_KS2, np.uint32(_KS0 + np.uint32(2))),
    (_KS0, np.uint32(_KS1 + np.uint32(3))),
    (_KS1, np.uint32(_KS2 + np.uint32(4))),
    (_KS2, np.uint32(_KS0 + np.uint32(5))),
)
_TINY = np.float32(np.finfo(np.float32).tiny)

_NC = 2
_NS = 16
_NW = _NC * _NS
_L = 16
_CHUNK = 120


def _rotl_i32(v, r):
    return lax.shift_left(v, np.int32(r)) | lax.shift_right_logical(
        v, np.int32(32 - r))


def _threefry_bits(cnt):
    x1 = cnt + np.int32(_KS1)
    x0 = x1
    first = True
    for g in range(5):
        rots = _ROT_A if g % 2 == 0 else _ROT_B
        for r in rots:
            if first:
                first = False
            else:
                x0 = x0 + x1
            x1 = _rotl_i32(x1, r)
            x1 = x1 ^ x0
        a, b = _INJ[g]
        x0 = x0 + np.int32(a)
        x1 = x1 + np.int32(b)
    return x0 ^ x1


def _sample_body(n, K, x_ref, e_ref, s_ref):
    i = pl.program_id(0)
    x = x_ref[...]
    e = e_ref[...]
    prod = lax.dot_general(x, e, (((1,), (1,)), ((), ())),
                           preferred_element_type=jnp.float32)
    xn = jnp.sum(x * x, axis=1, keepdims=True)
    ones = jnp.ones((1, _D), jnp.float32)
    en = lax.dot_general(ones, e * e, (((1,), (1,)), ((), ())),
                         preferred_element_type=jnp.float32)
    logits = -((xn + en - (prod + prod)) * np.float32(1.0 / K))

    col = lax.broadcasted_iota(jnp.int32, (_TILE, K), 1)
    row = lax.broadcasted_iota(jnp.int32, (_TILE, K), 0)
    base = i * np.int32(_TILE * K) + row * np.int32(K) + col

    idxs = []
    for s in range(_S):
        cnt = base + np.int32(s * n * K)
        bits = _threefry_bits(cnt)
        fb = lax.shift_right_logical(bits, np.int32(9)) | np.int32(0x3F800000)
        u = lax.bitcast_convert_type(fb, jnp.float32) - np.float32(1.0)
        w = jnp.log(-jnp.log(u)) - logits
        mn = jnp.min(w, axis=1, keepdims=True)
        mask = w == mn
        idx = jnp.min(jnp.where(mask, col, np.int32(K)), axis=1)
        idxs.append(idx[:, None])
    s_ref[...] = jnp.concatenate(idxs, axis=1)


def _tc_sample(x, embeds):
    n = x.shape[0]
    K = embeds.shape[0]
    body = functools.partial(_sample_body, n, K)
    return pl.pallas_call(
        body,
        grid=(n // _TILE,),
        in_specs=[
            pl.BlockSpec((_TILE, _D), lambda i: (i, 0)),
            pl.BlockSpec((K, _D), lambda i: (0, 0)),
        ],
        out_specs=pl.BlockSpec((_TILE, _S), lambda i: (i, 0)),
        out_shape=jax.ShapeDtypeStruct((n, _S), jnp.int32),
        compiler_params=pltpu.CompilerParams(
            dimension_semantics=("parallel",)),
    )(x, embeds)


def _sc_gather_mean(x, table128, samples_idx):
    n = x.shape[0]
    tok_per_w = n // _NW
    b_per_w = tok_per_w * _S
    n_chunks = b_per_w // _CHUNK
    blk_chunks = 4
    blk_samp = blk_chunks * _CHUNK
    blk_tok = blk_samp // _S
    n_blocks = n_chunks // blk_chunks
    mesh = plsc.VectorSubcoreMesh(core_axis_name="c", subcore_axis_name="s")

    @functools.partial(
        pl.kernel, mesh=mesh,
        out_type=jax.ShapeDtypeStruct((n, _D), jnp.float32),
        scratch_types=[
            pltpu.VMEM((n_chunks, _CHUNK), jnp.int32),
            pltpu.VMEM((blk_samp, 2 * _D), jnp.float32),
            pltpu.VMEM((tok_per_w, _D), jnp.float32),
            pltpu.SemaphoreType.DMA,
        ],
    )
    def k(x_hbm, table_hbm, idx_hbm, out_hbm, idx_v, rows_v, xq_v, sem):
        wid = lax.axis_index("s") * _NC + lax.axis_index("c")
        tbase = wid * tok_per_w
        pltpu.sync_copy(idx_hbm.at[wid], idx_v)
        pltpu.sync_copy(x_hbm.at[pl.ds(tbase, tok_per_w)], xq_v)
        for b in range(n_blocks):
            for j in range(blk_chunks):
                pltpu.async_copy(
                    table_hbm.at[idx_v.at[b * blk_chunks + j]],
                    rows_v.at[pl.ds(j * _CHUNK, _CHUNK)], sem)
            for j in range(blk_chunks):
                pltpu.make_async_copy(
                    table_hbm.at[idx_v.at[b * blk_chunks + j]],
                    rows_v.at[pl.ds(j * _CHUNK, _CHUNK)], sem).wait()

            def body(t, carry):
                for vv in range(_D // _L):
                    sl = pl.ds(vv * _L, _L)
                    acc = rows_v[t * _S + 0, sl]
                    for s in range(1, _S):
                        acc = acc + rows_v[t * _S + s, sl]
                    q = acc / np.float32(_S)
                    tt = b * blk_tok + t
                    xv = xq_v[tt, sl]
                    xq_v[tt, sl] = xv + (q - xv)
                return carry

            lax.fori_loop(0, blk_tok, body, 0)
        pltpu.sync_copy(xq_v, out_hbm.at[pl.ds(tbase, tok_per_w)])

    return k(x, table128, samples_idx)


def kernel(inputs, embeds):
    size = inputs.shape
    x = inputs.reshape(-1, _D)
    samples = _tc_sample(x, embeds)
    table128 = jnp.pad(embeds, ((0, 0), (0, _D)))
    q = _sc_gather_mean(x, table128,
                        samples.reshape(_NW, -1, _CHUNK))
    return (q.reshape(size), samples.reshape(size[:-1] + (_S,)))
